# Initial kernel scaffold; baseline (speedup 1.0000x reference)
import jax
import jax.numpy as jnp
from jax.experimental import pallas as pl


def kernel(feats, edge_index, q, pos, qW0, qb0, qWs, qbs, W0, b0, Ws, bs, fWs, fbs, q_atts, atts, fq_att, f_att, mlpW1, mlpb1, mlpW2, mlpb2, lqW, lqb, lfW, lfb):
    raise NotImplementedError("write your pallas kernel here")


# speedup vs baseline: 11.9577x; 11.9577x over previous
import functools

import jax
import jax.numpy as jnp
from jax import lax
from jax.experimental import pallas as pl
from jax.experimental.pallas import tpu as pltpu
from jax.experimental.pallas import tpu_sc as plsc

N = 10000
NP = 10240
E = 320000
H = 128
TAU = 0.5

NC = 2
NS = 16
CHUNK = 125
NCH_DEG = E // (NC * NS) // CHUNK
NCH = E // NS // CHUNK
HW = 96
NG = 4
DW = 16
RPT = NP // NS
RB = 1000
GRID = N // RB

_mesh = plsc.VectorSubcoreMesh(core_axis_name="c", subcore_axis_name="s")
_sc_params = pltpu.CompilerParams(use_tc_tiling_on_sc=False)


@functools.partial(
    pl.kernel,
    out_type=jax.ShapeDtypeStruct((NC, NP, DW), jnp.float32),
    mesh=_mesh,
    scratch_types=[
        pltpu.VMEM((NCH_DEG, CHUNK), jnp.int32),
        pltpu.VMEM((CHUNK, DW), jnp.float32),
        pltpu.VMEM_SHARED((NP, DW), jnp.float32),
    ],
    compiler_params=_sc_params,
)
def _sc_degree(dstD, ones_hbm, zcol, out, dst_v, ones_v, acc):
    cid = lax.axis_index("c")
    sid = lax.axis_index("s")
    w = cid * NS + sid
    pltpu.sync_copy(dstD.at[w], dst_v)
    pltpu.sync_copy(ones_hbm, ones_v)
    r0 = sid * RPT
    pltpu.sync_copy(zcol.at[pl.ds(r0, RPT)], acc.at[pl.ds(r0, RPT)])
    plsc.subcore_barrier()

    def body(c, carry):
        pltpu.sync_copy(ones_v, acc.at[dst_v.at[c]], add=True)
        return carry

    lax.fori_loop(0, NCH_DEG, body, 0)
    plsc.subcore_barrier()
    pltpu.sync_copy(acc.at[pl.ds(r0, RPT)], out.at[cid, pl.ds(r0, RPT)])


@functools.partial(
    pl.kernel,
    out_type=jax.ShapeDtypeStruct((NG, NP, HW), jnp.float32),
    mesh=_mesh,
    scratch_types=[
        pltpu.VMEM((NCH, CHUNK), jnp.int32),
        pltpu.VMEM((NCH, CHUNK), jnp.int32),
        pltpu.VMEM((CHUNK, HW), jnp.float32),
        pltpu.VMEM_SHARED((NP, HW), jnp.float32),
        pltpu.SemaphoreType.DMA,
    ],
    compiler_params=_sc_params,
)
def _sc_scatter(y4, srcG, dstT, z, out, src_v, dst_v, gbuf, acc, sem):
    cid = lax.axis_index("c")
    sid = lax.axis_index("s")
    r0 = sid * RPT
    pltpu.sync_copy(dstT.at[sid], dst_v)
    for p in range(2):
        g = 2 * p + cid
        pltpu.sync_copy(srcG.at[g * NS + sid], src_v)
        pltpu.sync_copy(z.at[pl.ds(r0, RPT)], acc.at[pl.ds(r0, RPT)])
        plsc.subcore_barrier()

        def body(c, carry):
            pltpu.async_copy(y4.at[src_v.at[c]], gbuf, sem).wait()
            pltpu.sync_copy(gbuf, acc.at[dst_v.at[c]], add=True)
            return carry

        lax.fori_loop(0, NCH, body, 0)
        plsc.subcore_barrier()
        pltpu.sync_copy(acc.at[pl.ds(r0, RPT)], out.at[g, pl.ds(r0, RPT)])


def _full(shape):
    return pl.BlockSpec(shape, lambda i: tuple(0 for _ in shape))


def _rows(width):
    return pl.BlockSpec((RB, width), lambda i: (i, 0))


def _srows(lead, width):
    return pl.BlockSpec((lead, RB, width), lambda i: (0, i, 0))


def _dinv_of(degR):
    d = degR[...]
    return lax.rsqrt(d[0, :, 0:1] + d[1, :, 0:1] + 1.0)


def _fuse2(a, b, waT, wbT):
    sa = jnp.sum(a * waT, axis=1, keepdims=True)
    sb = jnp.sum(b * wbT, axis=1, keepdims=True)
    m = jnp.maximum(sa, sb)
    ea = jnp.exp(sa - m)
    eb = jnp.exp(sb - m)
    return (ea * a + eb * b) / (ea + eb)


def _scat(sp):
    s = sp[...]
    return jnp.concatenate([s[0], s[1], s[2], s[3]], axis=1)


def _tc_prep0_body(feats, degR, isq, qW0, W0, lqW, lqb, lfW, lfb,
                   fqaT, faT, fW0, y_out):
    dinv = _dinv_of(degR)
    iq = isq[...]
    ya = dinv * (iq * qW0[...])
    yb = dinv * jnp.dot(feats[...], W0[...], preferred_element_type=jnp.float32)
    q2 = iq * lqW[...] + lqb[...]
    f2 = jnp.dot(feats[...], lfW[...], preferred_element_type=jnp.float32) + lfb[...]
    hf_ = _fuse2(q2, f2, fqaT[...], faT[...])
    yc = dinv * jnp.dot(hf_, fW0[...], preferred_element_type=jnp.float32)
    y_out[...] = jnp.concatenate([ya, yb, yc], axis=1)


def _tc_mid_body(y, sp, degR, qW, Wm, fW, qb, bm, fb, qaT, aT, y_out):
    dinv = _dinv_of(degR)
    yv = y[...]
    sv = _scat(sp)
    hq = jnp.maximum(dinv * (sv[:, 0:H] + yv[:, 0:H]) + qb[...], 0.0)
    h = jnp.maximum(dinv * (sv[:, H:2 * H] + yv[:, H:2 * H]) + bm[...], 0.0)
    gf = dinv * (sv[:, 2 * H:] + yv[:, 2 * H:]) + fb[...]
    hf = jnp.maximum(_fuse2(hq, h, qaT[...], aT[...]) + gf, 0.0)
    y_out[...] = jnp.concatenate([
        dinv * jnp.dot(hq, qW[...], preferred_element_type=jnp.float32),
        dinv * jnp.dot(h, Wm[...], preferred_element_type=jnp.float32),
        dinv * jnp.dot(hf, fW[...], preferred_element_type=jnp.float32),
    ], axis=1)


def _tc_last_body(y, sp, degR, isq, qb, bm, fb, qaT, aT,
                  mlpW1, mlpb1, mlpW2, mlpb2, h_out, hv_out):
    i = pl.program_id(0)
    dinv = _dinv_of(degR)
    yv = y[...]
    sv = _scat(sp)
    hq = dinv * (sv[:, 0:H] + yv[:, 0:H]) + qb[...]
    h = dinv * (sv[:, H:2 * H] + yv[:, H:2 * H]) + bm[...]
    gf = dinv * (sv[:, 2 * H:] + yv[:, 2 * H:]) + fb[...]
    hf = _fuse2(hq, h, qaT[...], aT[...]) + gf
    t = jnp.maximum(jnp.dot(hf, mlpW1[...], preferred_element_type=jnp.float32)
                    + mlpb1[...], 0.0)
    hfin = jnp.dot(t, mlpW2[...], preferred_element_type=jnp.float32) + mlpb2[...]
    h_out[...] = hfin

    @pl.when(i == 0)
    def _():
        hv_out[...] = jnp.zeros_like(hv_out)

    hv_out[...] += jnp.sum(isq[...] * hfin, axis=0, keepdims=True)


def _tc_loss_body(h, hv, sel, part_out):
    i = pl.program_id(0)
    hval = h[...]
    hvv = hv[...]
    num = jnp.sum(hval * hvv, axis=1, keepdims=True)
    nh = jnp.sqrt(jnp.sum(hval * hval, axis=1, keepdims=True))
    nv = jnp.sqrt(jnp.sum(hvv * hvv))
    s = num / jnp.maximum(nh * nv, 1e-8) / TAU
    sim = jnp.exp(s)
    selv = sel[...]
    lane = lax.broadcasted_iota(jnp.int32, (1, H), 1)
    row = (jnp.where(lane == 0, jnp.sum(sim), 0.0)
           + jnp.where(lane == 1, jnp.sum(selv * s), 0.0)
           + jnp.where(lane == 2, jnp.sum(selv), 0.0))

    @pl.when(i == 0)
    def _():
        part_out[...] = jnp.zeros_like(part_out)

    part_out[...] += row


def _prep0(feats, degp, isq, qW0, W0, lqW, lqb, lfW, lfb, fqaT, faT, fW0):
    return pl.pallas_call(
        _tc_prep0_body,
        grid=(GRID,),
        in_specs=[_rows(H), _srows(NC, DW), _rows(1),
                  _full((1, H)), _full((H, H)), _full((1, H)), _full((1, H)),
                  _full((H, H)), _full((1, H)), _full((1, H)), _full((1, H)),
                  _full((H, H))],
        out_specs=_rows(3 * H),
        out_shape=jax.ShapeDtypeStruct((N, 3 * H), jnp.float32),
    )(feats, degp, isq, qW0, W0, lqW, lqb, lfW, lfb, fqaT, faT, fW0)


def _mid(y, sp, degp, qW, Wm, fW, qb, bm, fb, qaT, aT):
    return pl.pallas_call(
        _tc_mid_body,
        grid=(GRID,),
        in_specs=[_rows(3 * H), _srows(NG, HW), _srows(NC, DW),
                  _full((H, H)), _full((H, H)), _full((H, H)),
                  _full((1, H)), _full((1, H)), _full((1, H)),
                  _full((1, H)), _full((1, H))],
        out_specs=_rows(3 * H),
        out_shape=jax.ShapeDtypeStruct((N, 3 * H), jnp.float32),
    )(y, sp, degp, qW, Wm, fW, qb, bm, fb, qaT, aT)


def _last(y, sp, degp, isq, qb, bm, fb, qaT, aT, mlpW1, mlpb1, mlpW2, mlpb2):
    return pl.pallas_call(
        _tc_last_body,
        grid=(GRID,),
        in_specs=[_rows(3 * H), _srows(NG, HW), _srows(NC, DW), _rows(1),
                  _full((1, H)), _full((1, H)), _full((1, H)),
                  _full((1, H)), _full((1, H)),
                  _full((H, H)), _full((1, H)), _full((H, H)), _full((1, H))],
        out_specs=[_rows(H), pl.BlockSpec((1, H), lambda i: (0, 0))],
        out_shape=[jax.ShapeDtypeStruct((N, H), jnp.float32),
                   jax.ShapeDtypeStruct((1, H), jnp.float32)],
    )(y, sp, degp, isq, qb, bm, fb, qaT, aT, mlpW1, mlpb1, mlpW2, mlpb2)


def _loss_parts(h, hv, sel):
    return pl.pallas_call(
        _tc_loss_body,
        grid=(GRID,),
        in_specs=[_rows(H), _full((1, H)), _rows(1)],
        out_specs=pl.BlockSpec((1, H), lambda i: (0, 0)),
        out_shape=jax.ShapeDtypeStruct((1, H), jnp.float32),
    )(h, hv, sel)


def kernel(feats, edge_index, q, pos, qW0, qb0, qWs, qbs, W0, b0, Ws, bs,
           fWs, fbs, q_atts, atts, fq_att, f_att, mlpW1, mlpb1, mlpW2, mlpb2,
           lqW, lqb, lfW, lfb):
    f32 = jnp.float32
    src = edge_index[0].astype(jnp.int32)
    dst = edge_index[1].astype(jnp.int32)

    dstD = dst.reshape(NC * NS, NCH_DEG, CHUNK)
    src4 = jnp.concatenate([NG * src + g for g in range(NG)]).reshape(
        NG * NS, NCH, CHUNK)
    dstT = dst.reshape(NS, NCH, CHUNK)
    ones_c = jnp.ones((CHUNK, DW), f32)
    zcol = jnp.zeros((NP, DW), f32)
    zacc = jnp.zeros((NP, HW), f32)

    qi = jnp.asarray(q, jnp.int32)
    rows = jnp.arange(N, dtype=jnp.int32)
    isq = (rows == qi).astype(f32).reshape(N, 1)
    selb = jnp.zeros((N,), bool).at[pos].set(True).at[qi].set(False)
    sel = selb.astype(f32).reshape(N, 1)

    r1 = lambda v: v.reshape(1, H).astype(f32)
    qW0r, lqWr, lqbr, lfbr = r1(qW0), r1(lqW), r1(lqb), r1(lfb)
    qb0r, b0r = r1(qb0), r1(b0)
    fqaT, faT = r1(fq_att), r1(f_att)
    qaT = [r1(q_atts[k]) for k in range(3)]
    aT = [r1(atts[k]) for k in range(3)]
    fbr = [r1(fbs[k]) for k in range(3)]
    qbsr = [r1(qbs[k]) for k in range(2)]
    bsr = [r1(bs[k]) for k in range(2)]
    mlpb1r, mlpb2r = r1(mlpb1), r1(mlpb2)

    degp = _sc_degree(dstD, ones_c, zcol)

    y0 = _prep0(feats.astype(f32), degp, isq, qW0r, W0.astype(f32),
                lqWr, lqbr, lfW.astype(f32), lfbr, fqaT, faT, fWs[0].astype(f32))
    s0 = _sc_scatter(y0.reshape(NG * N, HW), src4, dstT, zacc)

    y1 = _mid(y0, s0, degp, qWs[0].astype(f32), Ws[0].astype(f32),
              fWs[1].astype(f32), qb0r, b0r, fbr[0], qaT[0], aT[0])
    s1 = _sc_scatter(y1.reshape(NG * N, HW), src4, dstT, zacc)

    y2 = _mid(y1, s1, degp, qWs[1].astype(f32), Ws[1].astype(f32),
              fWs[2].astype(f32), qbsr[0], bsr[0], fbr[1], qaT[1], aT[1])
    s2 = _sc_scatter(y2.reshape(NG * N, HW), src4, dstT, zacc)

    h_, hv = _last(y2, s2, degp, isq, qbsr[1], bsr[1], fbr[2],
                   qaT[2], aT[2], mlpW1.astype(f32), mlpb1r,
                   mlpW2.astype(f32), mlpb2r)

    p = _loss_parts(h_, hv, sel)[0]
    loss = -(p[1] - p[2] * jnp.log(p[0])) / p[2]
    return loss, h_
